# Initial kernel scaffold; baseline (speedup 1.0000x reference)
import jax
import jax.numpy as jnp
from jax.experimental import pallas as pl


def kernel(hidden_atom_emb, edge_emb, edge_list, W1, b1, W2, b2, linear_coef):
    raise NotImplementedError("write your pallas kernel here")


# speedup vs baseline: 2.2526x; 2.2526x over previous
import functools

import jax
import jax.numpy as jnp
from jax import lax
from jax.experimental import pallas as pl
from jax.experimental.pallas import tpu as pltpu
from jax.experimental.pallas import tpu_sc as plsc

N = 10000
E = 320000
D = 128
ED = 16
NC = 2
NS = 16
NW = NC * NS
EPW = E // NW
CH = 80
NCH = EPW // CH
NZCH = N // CH
NB = 80

_mesh = plsc.VectorSubcoreMesh(core_axis_name="c", subcore_axis_name="s")


@functools.partial(
    pl.kernel,
    mesh=_mesh,
    out_type=jax.ShapeDtypeStruct((E, D), jnp.float32),
    scratch_types=[
        pltpu.VMEM((CH,), jnp.int32),
        pltpu.VMEM((CH, D), jnp.float32),
        pltpu.SemaphoreType.DMA,
    ],
)
def _gather(table_hbm, col_hbm, out_hbm, idx_v, g_v, sem):
    wid = lax.axis_index("s") * NC + lax.axis_index("c")
    ebase = wid * EPW

    def body(j, carry):
        pltpu.sync_copy(col_hbm.at[wid, j, 0], idx_v)
        pltpu.async_copy(table_hbm.at[idx_v], g_v, sem).wait()
        pltpu.sync_copy(g_v, out_hbm.at[pl.ds(ebase + j * CH, CH)])
        return carry

    lax.fori_loop(0, NCH, body, 0)


def _mlp_body(e_ref, g_ref, r_ref, w1_ref, b1_ref, w2_ref, b2_ref, c_ref,
              m_ref, aoh_ref):
    h = jnp.tanh(
        jnp.dot(e_ref[...], w1_ref[...], preferred_element_type=jnp.float32)
        + b1_ref[...]
    )
    f = jnp.dot(h, w2_ref[...], preferred_element_type=jnp.float32) + b2_ref[...]
    hce = g_ref[...] * f
    attn = jnp.exp(jnp.sum(hce * c_ref[...], axis=1, keepdims=True))
    m_ref[...] = hce * attn
    lanes = lax.broadcasted_iota(jnp.int32, m_ref.shape, 1)
    onehot = (lanes == (r_ref[...] & 127)).astype(jnp.float32)
    aoh_ref[...] = attn * onehot


def _mlp(edge_emb, g, row2d, W1, b1r, W2, b2r, cr):
    be = 1000
    grid = E // be
    return pl.pallas_call(
        _mlp_body,
        grid=(grid,),
        in_specs=[
            pl.BlockSpec((be, ED), lambda i: (i, 0)),
            pl.BlockSpec((be, D), lambda i: (i, 0)),
            pl.BlockSpec((be, 1), lambda i: (i, 0)),
            pl.BlockSpec((ED, D), lambda i: (0, 0)),
            pl.BlockSpec((1, D), lambda i: (0, 0)),
            pl.BlockSpec((D, D), lambda i: (0, 0)),
            pl.BlockSpec((1, D), lambda i: (0, 0)),
            pl.BlockSpec((1, D), lambda i: (0, 0)),
        ],
        out_specs=[
            pl.BlockSpec((be, D), lambda i: (i, 0)),
            pl.BlockSpec((be, D), lambda i: (i, 0)),
        ],
        out_shape=[
            jax.ShapeDtypeStruct((E, D), jnp.float32),
            jax.ShapeDtypeStruct((E, D), jnp.float32),
        ],
    )(edge_emb, g, row2d, W1, b1r, W2, b2r, cr)


@functools.partial(
    pl.kernel,
    mesh=_mesh,
    out_type=[
        jax.ShapeDtypeStruct((NC, N, D), jnp.float32),
        jax.ShapeDtypeStruct((NC, NB, D), jnp.float32),
    ],
    scratch_types=[
        pltpu.VMEM((CH,), jnp.int32),
        pltpu.VMEM((CH,), jnp.int32),
        pltpu.VMEM((CH, D), jnp.float32),
        pltpu.VMEM((CH, D), jnp.float32),
        pltpu.VMEM_SHARED((N, D), jnp.float32),
        pltpu.VMEM_SHARED((NB, D), jnp.float32),
    ],
)
def _scatter(m_hbm, aoh_hbm, row_hbm, num_out, asum_out,
             idx_v, idx2_v, m_v, aoh_v, num_s, as_s):
    cid = lax.axis_index("c")
    sid = lax.axis_index("s")
    wid = sid * NC + cid
    ebase = wid * EPW

    zero16 = jnp.zeros((16,), jnp.float32)

    def zm(i, carry):
        m_v[i // 8, pl.ds((i % 8) * 16, 16)] = zero16
        return carry

    lax.fori_loop(0, CH * 8, zm, 0)

    for t in range(8):
        c = sid + t * NS

        @pl.when(c < NZCH)
        def _():
            pltpu.sync_copy(m_v, num_s.at[pl.ds(c * CH, CH)])

    @pl.when(sid == 0)
    def _():
        pltpu.sync_copy(m_v, as_s)

    plsc.subcore_barrier()

    def body(j, carry):
        pltpu.sync_copy(row_hbm.at[wid, j, 0], idx_v)
        pltpu.sync_copy(m_hbm.at[pl.ds(ebase + j * CH, CH)], m_v)
        pltpu.sync_copy(aoh_hbm.at[pl.ds(ebase + j * CH, CH)], aoh_v)
        for k in range(CH // 16):
            iv = idx_v[pl.ds(k * 16, 16)]
            idx2_v[pl.ds(k * 16, 16)] = lax.shift_right_logical(iv, 7)
        pltpu.sync_copy(m_v, num_s.at[idx_v], add=True)
        pltpu.sync_copy(aoh_v, as_s.at[idx2_v], add=True)
        return carry

    lax.fori_loop(0, NCH, body, 0)
    plsc.subcore_barrier()

    @pl.when(sid == 0)
    def _():
        pltpu.sync_copy(as_s, asum_out.at[cid])

    for t in range(8):
        c = sid + t * NS

        @pl.when(c < NZCH)
        def _():
            r = c * CH
            pltpu.sync_copy(num_s.at[pl.ds(r, CH)], num_out.at[cid, pl.ds(r, CH)])


def _norm_body(num_ref, asum_ref, out_ref):
    nm = num_ref[0] + num_ref[1]
    norm = jnp.sum(asum_ref[...], axis=0)
    safe = jnp.where(norm > 0, norm, 1.0)
    out_ref[...] = jnp.where(norm > 0, nm / safe, nm)


def _norm(num_p, asum_p):
    bn = 1000
    return pl.pallas_call(
        _norm_body,
        grid=(N // bn,),
        in_specs=[
            pl.BlockSpec((NC, bn, D), lambda i: (0, i, 0)),
            pl.BlockSpec((NC, bn, 1), lambda i: (0, i, 0)),
        ],
        out_specs=pl.BlockSpec((bn, D), lambda i: (i, 0)),
        out_shape=jax.ShapeDtypeStruct((N, D), jnp.float32),
    )(num_p, asum_p)


def kernel(hidden_atom_emb, edge_emb, edge_list, W1, b1, W2, b2, linear_coef):
    col3 = edge_list[:, 1].astype(jnp.int32).reshape(NW, NCH, 1, CH)
    row3 = edge_list[:, 0].astype(jnp.int32).reshape(NW, NCH, 1, CH)
    g = _gather(hidden_atom_emb, col3)
    m, aoh = _mlp(
        edge_emb, g, edge_list[:, 0:1].astype(jnp.int32), W1,
        b1.reshape(1, D), W2, b2.reshape(1, D), linear_coef.reshape(1, D),
    )
    num_p, asum_p = _scatter(m, aoh, row3)
    asum = asum_p.reshape(NC, NB * D)[:, :N].reshape(NC, N, 1)
    return _norm(num_p, asum)


# speedup vs baseline: 2.9990x; 1.3313x over previous
import functools

import jax
import jax.numpy as jnp
from jax import lax
from jax.experimental import pallas as pl
from jax.experimental.pallas import tpu as pltpu
from jax.experimental.pallas import tpu_sc as plsc

N = 10000
E = 320000
D = 128
ED = 16
NC = 2
NS = 16
NW = NC * NS
EPW = E // NW
CH = 80
NCH = EPW // CH
NZCH = N // CH
NB = 80
NBUF = 5
NSB = 2

_mesh = plsc.VectorSubcoreMesh(core_axis_name="c", subcore_axis_name="s")


@functools.partial(
    pl.kernel,
    mesh=_mesh,
    out_type=jax.ShapeDtypeStruct((E, D), jnp.float32),
    scratch_types=(
        [pltpu.VMEM((CH,), jnp.int32) for _ in range(NBUF)]
        + [pltpu.VMEM((CH, D), jnp.float32) for _ in range(NBUF)]
        + [pltpu.SemaphoreType.DMA] * 3
    ),
)
def _gather(table_hbm, col_hbm, out_hbm, *rest):
    idx_b = rest[:NBUF]
    g_b = rest[NBUF:2 * NBUF]
    sem_i, sem_g, sem_w = rest[2 * NBUF:]
    wid = lax.axis_index("s") * NC + lax.axis_index("c")
    ebase = wid * EPW

    def body(jo, carry):
        j0 = jo * NBUF
        ih = [pltpu.async_copy(col_hbm.at[wid, j0 + b, 0], idx_b[b], sem_i)
              for b in range(NBUF)]
        gh = []
        for b in range(NBUF):
            ih[b].wait()
            gh.append(pltpu.async_copy(table_hbm.at[idx_b[b]], g_b[b], sem_g))
        wh = []
        for b in range(NBUF):
            gh[b].wait()
            wh.append(pltpu.async_copy(
                g_b[b], out_hbm.at[pl.ds(ebase + (j0 + b) * CH, CH)], sem_w))
        for b in range(NBUF):
            wh[b].wait()
        return carry

    lax.fori_loop(0, NCH // NBUF, body, 0)


def _mlp_body(e_ref, g_ref, r_ref, w1_ref, b1_ref, w2_ref, b2_ref, c_ref,
              m_ref, aoh_ref):
    h = jnp.tanh(
        jnp.dot(e_ref[...], w1_ref[...], preferred_element_type=jnp.float32)
        + b1_ref[...]
    )
    f = jnp.dot(h, w2_ref[...], preferred_element_type=jnp.float32) + b2_ref[...]
    hce = g_ref[...] * f
    attn = jnp.exp(jnp.sum(hce * c_ref[...], axis=1, keepdims=True))
    m_ref[...] = hce * attn
    lanes = lax.broadcasted_iota(jnp.int32, m_ref.shape, 1)
    onehot = (lanes == (r_ref[...] & 127)).astype(jnp.float32)
    aoh_ref[...] = attn * onehot


def _mlp(edge_emb, g, row2d, W1, b1r, W2, b2r, cr):
    be = 1000
    grid = E // be
    return pl.pallas_call(
        _mlp_body,
        grid=(grid,),
        in_specs=[
            pl.BlockSpec((be, ED), lambda i: (i, 0)),
            pl.BlockSpec((be, D), lambda i: (i, 0)),
            pl.BlockSpec((be, 1), lambda i: (i, 0)),
            pl.BlockSpec((ED, D), lambda i: (0, 0)),
            pl.BlockSpec((1, D), lambda i: (0, 0)),
            pl.BlockSpec((D, D), lambda i: (0, 0)),
            pl.BlockSpec((1, D), lambda i: (0, 0)),
            pl.BlockSpec((1, D), lambda i: (0, 0)),
        ],
        out_specs=[
            pl.BlockSpec((be, D), lambda i: (i, 0)),
            pl.BlockSpec((be, D), lambda i: (i, 0)),
        ],
        out_shape=[
            jax.ShapeDtypeStruct((E, D), jnp.float32),
            jax.ShapeDtypeStruct((E, D), jnp.float32),
        ],
    )(edge_emb, g, row2d, W1, b1r, W2, b2r, cr)


@functools.partial(
    pl.kernel,
    mesh=_mesh,
    out_type=[
        jax.ShapeDtypeStruct((NC, N, D), jnp.float32),
        jax.ShapeDtypeStruct((NC, NB, D), jnp.float32),
    ],
    scratch_types=(
        [pltpu.VMEM((CH,), jnp.int32) for _ in range(2 * NSB)]
        + [pltpu.VMEM((CH, D), jnp.float32) for _ in range(2 * NSB)]
        + [pltpu.VMEM_SHARED((N, D), jnp.float32),
           pltpu.VMEM_SHARED((NB, D), jnp.float32)]
        + [pltpu.SemaphoreType.DMA] * (4 * NSB)
    ),
)
def _scatter(m_hbm, aoh_hbm, row_hbm, num_out, asum_out, *rest):
    idx_b = rest[:NSB]
    idx2_b = rest[NSB:2 * NSB]
    m_b = rest[2 * NSB:3 * NSB]
    aoh_b = rest[3 * NSB:4 * NSB]
    num_s, as_s = rest[4 * NSB:4 * NSB + 2]
    sems = rest[4 * NSB + 2:]
    sem_i = sems[:NSB]
    sem_m = sems[NSB:2 * NSB]
    sem_a = sems[2 * NSB:3 * NSB]
    sem_s = sems[3 * NSB:]
    cid = lax.axis_index("c")
    sid = lax.axis_index("s")
    wid = sid * NC + cid
    ebase = wid * EPW

    zero16 = jnp.zeros((16,), jnp.float32)

    def zm(i, carry):
        m_b[0][i // 8, pl.ds((i % 8) * 16, 16)] = zero16
        return carry

    lax.fori_loop(0, CH * 8, zm, 0)

    for t in range(8):
        c = sid + t * NS

        @pl.when(c < NZCH)
        def _():
            pltpu.sync_copy(m_b[0], num_s.at[pl.ds(c * CH, CH)])

    @pl.when(sid == 0)
    def _():
        pltpu.sync_copy(m_b[0], as_s)

    plsc.subcore_barrier()

    def _start_loads(b, j):
        pltpu.async_copy(row_hbm.at[wid, j, 0], idx_b[b], sem_i[b])
        pltpu.async_copy(m_hbm.at[pl.ds(ebase + j * CH, CH)], m_b[b], sem_m[b])
        pltpu.async_copy(aoh_hbm.at[pl.ds(ebase + j * CH, CH)], aoh_b[b],
                         sem_a[b])

    def _wait_loads(b, j):
        pltpu.make_async_copy(row_hbm.at[wid, j, 0], idx_b[b], sem_i[b]).wait()
        pltpu.make_async_copy(m_hbm.at[pl.ds(ebase + j * CH, CH)], m_b[b],
                              sem_m[b]).wait()
        pltpu.make_async_copy(aoh_hbm.at[pl.ds(ebase + j * CH, CH)], aoh_b[b],
                              sem_a[b]).wait()

    def _process(b):
        for k in range(CH // 16):
            iv = idx_b[b][pl.ds(k * 16, 16)]
            idx2_b[b][pl.ds(k * 16, 16)] = lax.shift_right_logical(iv, 7)
        hm = pltpu.async_copy(m_b[b], num_s.at[idx_b[b]], sem_s[b], add=True)
        ha = pltpu.async_copy(aoh_b[b], as_s.at[idx2_b[b]], sem_s[b], add=True)
        hm.wait()
        ha.wait()

    for b in range(NSB):
        _start_loads(b, b)

    def body(jo, carry):
        j = jo * NSB
        for b in range(NSB):
            _wait_loads(b, j + b)
            _process(b)

            @pl.when(j + b + NSB < NCH)
            def _():
                _start_loads(b, j + b + NSB)

        return carry

    lax.fori_loop(0, NCH // NSB, body, 0)
    _wait_loads(0, NCH - 1)
    _process(0)
    plsc.subcore_barrier()

    @pl.when(sid == 0)
    def _():
        pltpu.sync_copy(as_s, asum_out.at[cid])

    for t in range(8):
        c = sid + t * NS

        @pl.when(c < NZCH)
        def _():
            r = c * CH
            pltpu.sync_copy(num_s.at[pl.ds(r, CH)], num_out.at[cid, pl.ds(r, CH)])


def _norm_body(num_ref, asum_ref, out_ref):
    nm = num_ref[0] + num_ref[1]
    norm = jnp.sum(asum_ref[...], axis=0)
    safe = jnp.where(norm > 0, norm, 1.0)
    out_ref[...] = jnp.where(norm > 0, nm / safe, nm)


def _norm(num_p, asum_p):
    bn = 1000
    return pl.pallas_call(
        _norm_body,
        grid=(N // bn,),
        in_specs=[
            pl.BlockSpec((NC, bn, D), lambda i: (0, i, 0)),
            pl.BlockSpec((NC, bn, 1), lambda i: (0, i, 0)),
        ],
        out_specs=pl.BlockSpec((bn, D), lambda i: (i, 0)),
        out_shape=jax.ShapeDtypeStruct((N, D), jnp.float32),
    )(num_p, asum_p)


def kernel(hidden_atom_emb, edge_emb, edge_list, W1, b1, W2, b2, linear_coef):
    col3 = edge_list[:, 1].astype(jnp.int32).reshape(NW, NCH, 1, CH)
    row3 = edge_list[:, 0].astype(jnp.int32).reshape(NW, NCH, 1, CH)
    g = _gather(hidden_atom_emb, col3)
    m, aoh = _mlp(
        edge_emb, g, edge_list[:, 0:1].astype(jnp.int32), W1,
        b1.reshape(1, D), W2, b2.reshape(1, D), linear_coef.reshape(1, D),
    )
    num_p, asum_p = _scatter(m, aoh, row3)
    asum = asum_p.reshape(NC, NB * D)[:, :N].reshape(NC, N, 1)
    return _norm(num_p, asum)


# speedup vs baseline: 3.2385x; 1.0799x over previous
import functools

import jax
import jax.numpy as jnp
from jax import lax
from jax.experimental import pallas as pl
from jax.experimental.pallas import tpu as pltpu
from jax.experimental.pallas import tpu_sc as plsc

N = 10000
E = 320000
D = 128
ED = 16
NC = 2
NS = 16
NW = NC * NS
EPW = E // NW
CH = 80
NCH = EPW // CH
NZCH = N // CH
NB = 80
NGB = 4
NSB = 2

_mesh = plsc.VectorSubcoreMesh(core_axis_name="c", subcore_axis_name="s")


@functools.partial(
    pl.kernel,
    mesh=_mesh,
    out_type=jax.ShapeDtypeStruct((E, D), jnp.float32),
    scratch_types=(
        [pltpu.VMEM((CH,), jnp.int32) for _ in range(NGB)]
        + [pltpu.VMEM((CH, D), jnp.float32) for _ in range(NGB)]
        + [pltpu.VMEM_SHARED((N, D), jnp.float32)]
        + [pltpu.SemaphoreType.DMA] * (3 * NGB)
    ),
)
def _gather(table_hbm, col_hbm, out_hbm, *rest):
    idx_b = rest[:NGB]
    g_b = rest[NGB:2 * NGB]
    table_s = rest[2 * NGB]
    sems = rest[2 * NGB + 1:]
    sem_i = sems[:NGB]
    sem_g = sems[NGB:2 * NGB]
    sem_w = sems[2 * NGB:]
    sid = lax.axis_index("s")
    wid = sid * NC + lax.axis_index("c")
    ebase = wid * EPW

    for t in range(8):
        c = sid + t * NS

        @pl.when(c < NZCH)
        def _():
            pltpu.sync_copy(table_hbm.at[pl.ds(c * CH, CH)],
                            table_s.at[pl.ds(c * CH, CH)])

    plsc.subcore_barrier()

    def _start_idx(b, j):
        pltpu.async_copy(col_hbm.at[wid, j, 0], idx_b[b], sem_i[b])

    def _wait_idx(b, j):
        pltpu.make_async_copy(col_hbm.at[wid, j, 0], idx_b[b], sem_i[b]).wait()

    def _wait_write(b):
        pltpu.make_async_copy(g_b[b], out_hbm.at[pl.ds(ebase, CH)],
                              sem_w[b]).wait()

    for b in range(NGB):
        _start_idx(b, b)

    def body(jo, carry):
        j = jo * NGB
        for b in range(NGB):
            jj = j + b

            @pl.when(jj >= NGB)
            def _():
                _wait_write(b)

            _wait_idx(b, jj)
            pltpu.async_copy(table_s.at[idx_b[b]], g_b[b], sem_g[b]).wait()
            pltpu.async_copy(g_b[b], out_hbm.at[pl.ds(ebase + jj * CH, CH)],
                             sem_w[b])

            @pl.when(jj + NGB < NCH)
            def _():
                _start_idx(b, jj + NGB)

        return carry

    lax.fori_loop(0, NCH // NGB, body, 0)
    _wait_write(0)
    _wait_idx(0, NCH - 1)
    pltpu.async_copy(table_s.at[idx_b[0]], g_b[0], sem_g[0]).wait()
    pltpu.async_copy(g_b[0], out_hbm.at[pl.ds(ebase + (NCH - 1) * CH, CH)],
                     sem_w[0])
    for b in range(NGB):
        _wait_write(b)


def _mlp_body(e_ref, g_ref, r_ref, w1_ref, b1_ref, w2_ref, b2_ref, c_ref,
              m_ref, aoh_ref):
    h = jnp.tanh(
        jnp.dot(e_ref[...], w1_ref[...], preferred_element_type=jnp.float32)
        + b1_ref[...]
    )
    f = jnp.dot(h, w2_ref[...], preferred_element_type=jnp.float32) + b2_ref[...]
    hce = g_ref[...] * f
    attn = jnp.exp(jnp.sum(hce * c_ref[...], axis=1, keepdims=True))
    m_ref[...] = hce * attn
    lanes = lax.broadcasted_iota(jnp.int32, m_ref.shape, 1)
    onehot = (lanes == (r_ref[...] & 127)).astype(jnp.float32)
    aoh_ref[...] = attn * onehot


def _mlp(edge_emb, g, row2d, W1, b1r, W2, b2r, cr):
    be = 1000
    grid = E // be
    return pl.pallas_call(
        _mlp_body,
        grid=(grid,),
        in_specs=[
            pl.BlockSpec((be, ED), lambda i: (i, 0)),
            pl.BlockSpec((be, D), lambda i: (i, 0)),
            pl.BlockSpec((be, 1), lambda i: (i, 0)),
            pl.BlockSpec((ED, D), lambda i: (0, 0)),
            pl.BlockSpec((1, D), lambda i: (0, 0)),
            pl.BlockSpec((D, D), lambda i: (0, 0)),
            pl.BlockSpec((1, D), lambda i: (0, 0)),
            pl.BlockSpec((1, D), lambda i: (0, 0)),
        ],
        out_specs=[
            pl.BlockSpec((be, D), lambda i: (i, 0)),
            pl.BlockSpec((be, D), lambda i: (i, 0)),
        ],
        out_shape=[
            jax.ShapeDtypeStruct((E, D), jnp.float32),
            jax.ShapeDtypeStruct((E, D), jnp.float32),
        ],
    )(edge_emb, g, row2d, W1, b1r, W2, b2r, cr)


@functools.partial(
    pl.kernel,
    mesh=_mesh,
    out_type=[
        jax.ShapeDtypeStruct((NC, N, D), jnp.float32),
        jax.ShapeDtypeStruct((NC, NB, D), jnp.float32),
    ],
    scratch_types=(
        [pltpu.VMEM((CH,), jnp.int32) for _ in range(2 * NSB)]
        + [pltpu.VMEM((CH, D), jnp.float32) for _ in range(2 * NSB)]
        + [pltpu.VMEM_SHARED((N, D), jnp.float32),
           pltpu.VMEM_SHARED((NB, D), jnp.float32)]
        + [pltpu.SemaphoreType.DMA] * (4 * NSB)
    ),
)
def _scatter(m_hbm, aoh_hbm, row_hbm, num_out, asum_out, *rest):
    idx_b = rest[:NSB]
    idx2_b = rest[NSB:2 * NSB]
    m_b = rest[2 * NSB:3 * NSB]
    aoh_b = rest[3 * NSB:4 * NSB]
    num_s, as_s = rest[4 * NSB:4 * NSB + 2]
    sems = rest[4 * NSB + 2:]
    sem_i = sems[:NSB]
    sem_m = sems[NSB:2 * NSB]
    sem_a = sems[2 * NSB:3 * NSB]
    sem_s = sems[3 * NSB:]
    cid = lax.axis_index("c")
    sid = lax.axis_index("s")
    wid = sid * NC + cid
    ebase = wid * EPW

    zero16 = jnp.zeros((16,), jnp.float32)

    def zm(i, carry):
        m_b[0][i // 8, pl.ds((i % 8) * 16, 16)] = zero16
        return carry

    lax.fori_loop(0, CH * 8, zm, 0)

    for t in range(8):
        c = sid + t * NS

        @pl.when(c < NZCH)
        def _():
            pltpu.sync_copy(m_b[0], num_s.at[pl.ds(c * CH, CH)])

    @pl.when(sid == 0)
    def _():
        pltpu.sync_copy(m_b[0], as_s)

    plsc.subcore_barrier()

    def _start_loads(b, j):
        pltpu.async_copy(row_hbm.at[wid, j, 0], idx_b[b], sem_i[b])
        pltpu.async_copy(m_hbm.at[pl.ds(ebase + j * CH, CH)], m_b[b], sem_m[b])
        pltpu.async_copy(aoh_hbm.at[pl.ds(ebase + j * CH, CH)], aoh_b[b],
                         sem_a[b])

    def _wait_loads(b, j):
        pltpu.make_async_copy(row_hbm.at[wid, j, 0], idx_b[b], sem_i[b]).wait()
        pltpu.make_async_copy(m_hbm.at[pl.ds(ebase + j * CH, CH)], m_b[b],
                              sem_m[b]).wait()
        pltpu.make_async_copy(aoh_hbm.at[pl.ds(ebase + j * CH, CH)], aoh_b[b],
                              sem_a[b]).wait()

    def _process(b):
        for k in range(CH // 16):
            iv = idx_b[b][pl.ds(k * 16, 16)]
            idx2_b[b][pl.ds(k * 16, 16)] = lax.shift_right_logical(iv, 7)
        hm = pltpu.async_copy(m_b[b], num_s.at[idx_b[b]], sem_s[b], add=True)
        ha = pltpu.async_copy(aoh_b[b], as_s.at[idx2_b[b]], sem_s[b], add=True)
        hm.wait()
        ha.wait()

    for b in range(NSB):
        _start_loads(b, b)

    def body(jo, carry):
        j = jo * NSB
        for b in range(NSB):
            _wait_loads(b, j + b)
            _process(b)

            @pl.when(j + b + NSB < NCH)
            def _():
                _start_loads(b, j + b + NSB)

        return carry

    lax.fori_loop(0, NCH // NSB, body, 0)
    _wait_loads(0, NCH - 1)
    _process(0)
    plsc.subcore_barrier()

    @pl.when(sid == 0)
    def _():
        pltpu.sync_copy(as_s, asum_out.at[cid])

    for t in range(8):
        c = sid + t * NS

        @pl.when(c < NZCH)
        def _():
            r = c * CH
            pltpu.sync_copy(num_s.at[pl.ds(r, CH)], num_out.at[cid, pl.ds(r, CH)])


def _norm_body(num_ref, asum_ref, out_ref):
    nm = num_ref[0] + num_ref[1]
    norm = jnp.sum(asum_ref[...], axis=0)
    safe = jnp.where(norm > 0, norm, 1.0)
    out_ref[...] = jnp.where(norm > 0, nm / safe, nm)


def _norm(num_p, asum_p):
    bn = 1000
    return pl.pallas_call(
        _norm_body,
        grid=(N // bn,),
        in_specs=[
            pl.BlockSpec((NC, bn, D), lambda i: (0, i, 0)),
            pl.BlockSpec((NC, bn, 1), lambda i: (0, i, 0)),
        ],
        out_specs=pl.BlockSpec((bn, D), lambda i: (i, 0)),
        out_shape=jax.ShapeDtypeStruct((N, D), jnp.float32),
    )(num_p, asum_p)


def kernel(hidden_atom_emb, edge_emb, edge_list, W1, b1, W2, b2, linear_coef):
    col3 = edge_list[:, 1].astype(jnp.int32).reshape(NW, NCH, 1, CH)
    row3 = edge_list[:, 0].astype(jnp.int32).reshape(NW, NCH, 1, CH)
    g = _gather(hidden_atom_emb, col3)
    m, aoh = _mlp(
        edge_emb, g, edge_list[:, 0:1].astype(jnp.int32), W1,
        b1.reshape(1, D), W2, b2.reshape(1, D), linear_coef.reshape(1, D),
    )
    num_p, asum_p = _scatter(m, aoh, row3)
    asum = asum_p.reshape(NC, NB * D)[:, :N].reshape(NC, N, 1)
    return _norm(num_p, asum)


# speedup vs baseline: 3.5379x; 1.0924x over previous
import functools

import jax
import jax.numpy as jnp
from jax import lax
from jax.experimental import pallas as pl
from jax.experimental.pallas import tpu as pltpu
from jax.experimental.pallas import tpu_sc as plsc

N = 10000
E = 320000
D = 128
ED = 16
NC = 2
NS = 16
NW = NC * NS
EPW = E // NW
CH = 80
NCH = EPW // CH
NZCH = N // CH
NB = 80
NGB = 4
NSB = 2

_mesh = plsc.VectorSubcoreMesh(core_axis_name="c", subcore_axis_name="s")


@functools.partial(
    pl.kernel,
    mesh=_mesh,
    out_type=jax.ShapeDtypeStruct((E, D), jnp.float32),
    scratch_types=(
        [pltpu.VMEM((CH,), jnp.int32) for _ in range(NGB)]
        + [pltpu.VMEM((CH, D), jnp.float32) for _ in range(NGB)]
        + [pltpu.VMEM_SHARED((N, D), jnp.float32)]
        + [pltpu.SemaphoreType.DMA] * (3 * NGB)
    ),
)
def _gather(table_hbm, col_hbm, out_hbm, *rest):
    idx_b = rest[:NGB]
    g_b = rest[NGB:2 * NGB]
    table_s = rest[2 * NGB]
    sems = rest[2 * NGB + 1:]
    sem_i = sems[:NGB]
    sem_g = sems[NGB:2 * NGB]
    sem_w = sems[2 * NGB:]
    sid = lax.axis_index("s")
    wid = sid * NC + lax.axis_index("c")
    ebase = wid * EPW

    for t in range(8):
        c = sid + t * NS

        @pl.when(c < NZCH)
        def _():
            pltpu.sync_copy(table_hbm.at[pl.ds(c * CH, CH)],
                            table_s.at[pl.ds(c * CH, CH)])

    plsc.subcore_barrier()

    def _start_idx(b, j):
        pltpu.async_copy(col_hbm.at[wid, j, 0], idx_b[b], sem_i[b])

    def _wait_idx(b, j):
        pltpu.make_async_copy(col_hbm.at[wid, j, 0], idx_b[b], sem_i[b]).wait()

    def _wait_write(b):
        pltpu.make_async_copy(g_b[b], out_hbm.at[pl.ds(ebase, CH)],
                              sem_w[b]).wait()

    for b in range(NGB):
        _start_idx(b, b)

    def body(jo, carry):
        j = jo * NGB
        for b in range(NGB):
            jj = j + b

            @pl.when(jj >= NGB)
            def _():
                _wait_write(b)

            _wait_idx(b, jj)
            pltpu.async_copy(table_s.at[idx_b[b]], g_b[b], sem_g[b]).wait()
            pltpu.async_copy(g_b[b], out_hbm.at[pl.ds(ebase + jj * CH, CH)],
                             sem_w[b])

            @pl.when(jj + NGB < NCH)
            def _():
                _start_idx(b, jj + NGB)

        return carry

    lax.fori_loop(0, NCH // NGB, body, 0)
    _wait_write(0)
    _wait_idx(0, NCH - 1)
    pltpu.async_copy(table_s.at[idx_b[0]], g_b[0], sem_g[0]).wait()
    pltpu.async_copy(g_b[0], out_hbm.at[pl.ds(ebase + (NCH - 1) * CH, CH)],
                     sem_w[0])
    for b in range(NGB):
        _wait_write(b)


def _mlp_body(e_ref, g_ref, w1_ref, b1_ref, w2_ref, b2_ref, c_ref,
              m_ref, a_ref):
    h = jnp.tanh(
        jnp.dot(e_ref[...], w1_ref[...], preferred_element_type=jnp.float32)
        + b1_ref[...]
    )
    f = jnp.dot(h, w2_ref[...], preferred_element_type=jnp.float32) + b2_ref[...]
    hce = g_ref[...] * f
    attn = jnp.exp(jnp.sum(hce * c_ref[...], axis=1, keepdims=True))
    m_ref[...] = hce * attn
    a_ref[...] = attn


def _mlp(edge_emb, g, W1, b1r, W2, b2r, cr):
    be = 1000
    grid = E // be
    return pl.pallas_call(
        _mlp_body,
        grid=(grid,),
        in_specs=[
            pl.BlockSpec((be, ED), lambda i: (i, 0)),
            pl.BlockSpec((be, D), lambda i: (i, 0)),
            pl.BlockSpec((ED, D), lambda i: (0, 0)),
            pl.BlockSpec((1, D), lambda i: (0, 0)),
            pl.BlockSpec((D, D), lambda i: (0, 0)),
            pl.BlockSpec((1, D), lambda i: (0, 0)),
            pl.BlockSpec((1, D), lambda i: (0, 0)),
        ],
        out_specs=[
            pl.BlockSpec((be, D), lambda i: (i, 0)),
            pl.BlockSpec((be, 1), lambda i: (i, 0)),
        ],
        out_shape=[
            jax.ShapeDtypeStruct((E, D), jnp.float32),
            jax.ShapeDtypeStruct((E, 1), jnp.float32),
        ],
    )(edge_emb, g, W1, b1r, W2, b2r, cr)


@functools.partial(
    pl.kernel,
    mesh=_mesh,
    out_type=[
        jax.ShapeDtypeStruct((NC, N, D), jnp.float32),
        jax.ShapeDtypeStruct((NC, NB, D), jnp.float32),
    ],
    scratch_types=(
        [pltpu.VMEM((CH,), jnp.int32) for _ in range(2 * NSB)]
        + [pltpu.VMEM((CH,), jnp.float32) for _ in range(NSB)]
        + [pltpu.VMEM((CH, D), jnp.float32) for _ in range(2 * NSB)]
        + [pltpu.VMEM_SHARED((N, D), jnp.float32),
           pltpu.VMEM_SHARED((NB, D), jnp.float32)]
        + [pltpu.SemaphoreType.DMA] * (4 * NSB)
    ),
)
def _scatter(m_hbm, at_hbm, row_hbm, num_out, asum_out, *rest):
    idx_b = rest[:NSB]
    idx2_b = rest[NSB:2 * NSB]
    at_b = rest[2 * NSB:3 * NSB]
    m_b = rest[3 * NSB:4 * NSB]
    aoh_b = rest[4 * NSB:5 * NSB]
    num_s, as_s = rest[5 * NSB:5 * NSB + 2]
    sems = rest[5 * NSB + 2:]
    sem_i = sems[:NSB]
    sem_m = sems[NSB:2 * NSB]
    sem_a = sems[2 * NSB:3 * NSB]
    sem_s = sems[3 * NSB:]
    cid = lax.axis_index("c")
    sid = lax.axis_index("s")
    wid = sid * NC + cid
    ebase = wid * EPW

    zero16 = jnp.zeros((16,), jnp.float32)

    def zm(i, carry):
        m_b[0][i // 8, pl.ds((i % 8) * 16, 16)] = zero16
        return carry

    lax.fori_loop(0, CH * 8, zm, 0)

    for t in range(8):
        c = sid + t * NS

        @pl.when(c < NZCH)
        def _():
            pltpu.sync_copy(m_b[0], num_s.at[pl.ds(c * CH, CH)])

    @pl.when(sid == 0)
    def _():
        pltpu.sync_copy(m_b[0], as_s)

    plsc.subcore_barrier()

    def _start_loads(b, j):
        pltpu.async_copy(row_hbm.at[wid, j, 0], idx_b[b], sem_i[b])
        pltpu.async_copy(m_hbm.at[pl.ds(ebase + j * CH, CH)], m_b[b], sem_m[b])
        pltpu.async_copy(at_hbm.at[wid, j, 0], at_b[b], sem_a[b])

    def _wait_loads(b, j):
        pltpu.make_async_copy(row_hbm.at[wid, j, 0], idx_b[b], sem_i[b]).wait()
        pltpu.make_async_copy(m_hbm.at[pl.ds(ebase + j * CH, CH)], m_b[b],
                              sem_m[b]).wait()
        pltpu.make_async_copy(at_hbm.at[wid, j, 0], at_b[b], sem_a[b]).wait()

    io = lax.iota(jnp.int32, 16)
    zv = jnp.zeros((16,), jnp.float32)

    def _process(b):
        for k in range(CH // 16):
            iv = idx_b[b][pl.ds(k * 16, 16)]
            idx2_b[b][pl.ds(k * 16, 16)] = lax.shift_right_logical(iv, 7)

        def onehot_rows(k, carry):
            iv = idx_b[b][pl.ds(k * 16, 16)]
            av = at_b[b][pl.ds(k * 16, 16)]
            p = lax.bitwise_and(iv, 127)
            for e in range(16):
                pe = jnp.full((16,), p[e], jnp.int32)
                ae = jnp.full((16,), av[e], jnp.float32)
                for q in range(8):
                    aoh_b[b][k * 16 + e, pl.ds(q * 16, 16)] = jnp.where(
                        io + (16 * q) == pe, ae, zv)
            return carry

        lax.fori_loop(0, CH // 16, onehot_rows, 0)
        hm = pltpu.async_copy(m_b[b], num_s.at[idx_b[b]], sem_s[b], add=True)
        ha = pltpu.async_copy(aoh_b[b], as_s.at[idx2_b[b]], sem_s[b], add=True)
        hm.wait()
        ha.wait()

    for b in range(NSB):
        _start_loads(b, b)

    def body(jo, carry):
        j = jo * NSB
        for b in range(NSB):
            _wait_loads(b, j + b)
            _process(b)

            @pl.when(j + b + NSB < NCH)
            def _():
                _start_loads(b, j + b + NSB)

        return carry

    lax.fori_loop(0, NCH // NSB, body, 0)
    _wait_loads(0, NCH - 1)
    _process(0)
    plsc.subcore_barrier()

    @pl.when(sid == 0)
    def _():
        pltpu.sync_copy(as_s, asum_out.at[cid])

    for t in range(8):
        c = sid + t * NS

        @pl.when(c < NZCH)
        def _():
            r = c * CH
            pltpu.sync_copy(num_s.at[pl.ds(r, CH)], num_out.at[cid, pl.ds(r, CH)])


def _norm_body(num_ref, asum_ref, out_ref):
    nm = num_ref[0] + num_ref[1]
    norm = jnp.sum(asum_ref[...], axis=0)
    safe = jnp.where(norm > 0, norm, 1.0)
    out_ref[...] = jnp.where(norm > 0, nm / safe, nm)


def _norm(num_p, asum_p):
    bn = 1000
    return pl.pallas_call(
        _norm_body,
        grid=(N // bn,),
        in_specs=[
            pl.BlockSpec((NC, bn, D), lambda i: (0, i, 0)),
            pl.BlockSpec((NC, bn, 1), lambda i: (0, i, 0)),
        ],
        out_specs=pl.BlockSpec((bn, D), lambda i: (i, 0)),
        out_shape=jax.ShapeDtypeStruct((N, D), jnp.float32),
    )(num_p, asum_p)


def kernel(hidden_atom_emb, edge_emb, edge_list, W1, b1, W2, b2, linear_coef):
    col3 = edge_list[:, 1].astype(jnp.int32).reshape(NW, NCH, 1, CH)
    row3 = edge_list[:, 0].astype(jnp.int32).reshape(NW, NCH, 1, CH)
    g = _gather(hidden_atom_emb, col3)
    m, attn = _mlp(
        edge_emb, g, W1,
        b1.reshape(1, D), W2, b2.reshape(1, D), linear_coef.reshape(1, D),
    )
    at3 = attn.reshape(NW, NCH, 1, CH)
    num_p, asum_p = _scatter(m, at3, row3)
    asum = asum_p.reshape(NC, NB * D)[:, :N].reshape(NC, N, 1)
    return _norm(num_p, asum)


# speedup vs baseline: 4.9493x; 1.3989x over previous
import functools

import jax
import jax.numpy as jnp
from jax import lax
from jax.experimental import pallas as pl
from jax.experimental.pallas import tpu as pltpu
from jax.experimental.pallas import tpu_sc as plsc

N = 10000
E = 320000
D = 128
ED = 16
NC = 2
NS = 16
NW = NC * NS
EPW = E // NW
CH = 80
NCH = EPW // CH
NZCH = N // CH
NB = 80
NGB = 4
NSB = 2

_mesh = plsc.VectorSubcoreMesh(core_axis_name="c", subcore_axis_name="s")


@functools.partial(
    pl.kernel,
    mesh=_mesh,
    out_type=jax.ShapeDtypeStruct((E, D), jnp.float32),
    scratch_types=(
        [pltpu.VMEM((CH,), jnp.int32) for _ in range(NGB)]
        + [pltpu.VMEM((CH, D), jnp.float32) for _ in range(NGB)]
        + [pltpu.VMEM_SHARED((N, D), jnp.float32)]
        + [pltpu.SemaphoreType.DMA] * (3 * NGB)
    ),
)
def _gather(table_hbm, col_hbm, out_hbm, *rest):
    idx_b = rest[:NGB]
    g_b = rest[NGB:2 * NGB]
    table_s = rest[2 * NGB]
    sems = rest[2 * NGB + 1:]
    sem_i = sems[:NGB]
    sem_g = sems[NGB:2 * NGB]
    sem_w = sems[2 * NGB:]
    sid = lax.axis_index("s")
    wid = sid * NC + lax.axis_index("c")
    ebase = wid * EPW

    for t in range(8):
        c = sid + t * NS

        @pl.when(c < NZCH)
        def _():
            pltpu.sync_copy(table_hbm.at[pl.ds(c * CH, CH)],
                            table_s.at[pl.ds(c * CH, CH)])

    plsc.subcore_barrier()

    def _start_idx(b, j):
        pltpu.async_copy(col_hbm.at[wid, j, 0], idx_b[b], sem_i[b])

    def _wait_idx(b, j):
        pltpu.make_async_copy(col_hbm.at[wid, j, 0], idx_b[b], sem_i[b]).wait()

    def _wait_write(b):
        pltpu.make_async_copy(g_b[b], out_hbm.at[pl.ds(ebase, CH)],
                              sem_w[b]).wait()

    for b in range(NGB):
        _start_idx(b, b)

    def body(jo, carry):
        j = jo * NGB
        for b in range(NGB):
            jj = j + b

            @pl.when(jj >= NGB)
            def _():
                _wait_write(b)

            _wait_idx(b, jj)
            pltpu.async_copy(table_s.at[idx_b[b]], g_b[b], sem_g[b]).wait()
            pltpu.async_copy(g_b[b], out_hbm.at[pl.ds(ebase + jj * CH, CH)],
                             sem_w[b])

            @pl.when(jj + NGB < NCH)
            def _():
                _start_idx(b, jj + NGB)

        return carry

    lax.fori_loop(0, NCH // NGB, body, 0)
    _wait_write(0)
    _wait_idx(0, NCH - 1)
    pltpu.async_copy(table_s.at[idx_b[0]], g_b[0], sem_g[0]).wait()
    pltpu.async_copy(g_b[0], out_hbm.at[pl.ds(ebase + (NCH - 1) * CH, CH)],
                     sem_w[0])
    for b in range(NGB):
        _wait_write(b)


def _mlp_body(e_ref, g_ref, w1_ref, b1_ref, w2_ref, b2_ref, c_ref,
              m_ref, a_ref):
    h = jnp.tanh(
        jnp.dot(e_ref[...], w1_ref[...], preferred_element_type=jnp.float32)
        + b1_ref[...]
    )
    f = jnp.dot(h, w2_ref[...], preferred_element_type=jnp.float32) + b2_ref[...]
    hce = g_ref[...] * f
    attn = jnp.exp(jnp.sum(hce * c_ref[...], axis=1, keepdims=True))
    m_ref[...] = hce * attn
    a_ref[...] = attn


def _mlp(edge_emb, g, W1, b1r, W2, b2r, cr):
    be = 8000
    grid = E // be
    return pl.pallas_call(
        _mlp_body,
        grid=(grid,),
        in_specs=[
            pl.BlockSpec((be, ED), lambda i: (i, 0)),
            pl.BlockSpec((be, D), lambda i: (i, 0)),
            pl.BlockSpec((ED, D), lambda i: (0, 0)),
            pl.BlockSpec((1, D), lambda i: (0, 0)),
            pl.BlockSpec((D, D), lambda i: (0, 0)),
            pl.BlockSpec((1, D), lambda i: (0, 0)),
            pl.BlockSpec((1, D), lambda i: (0, 0)),
        ],
        out_specs=[
            pl.BlockSpec((be, D), lambda i: (i, 0)),
            pl.BlockSpec((be, 1), lambda i: (i, 0)),
        ],
        out_shape=[
            jax.ShapeDtypeStruct((E, D), jnp.float32),
            jax.ShapeDtypeStruct((E, 1), jnp.float32),
        ],
    )(edge_emb, g, W1, b1r, W2, b2r, cr)


@functools.partial(
    pl.kernel,
    mesh=_mesh,
    out_type=[
        jax.ShapeDtypeStruct((NC, N, D), jnp.float32),
        jax.ShapeDtypeStruct((NC, 1, N), jnp.float32),
    ],
    scratch_types=(
        [pltpu.VMEM((CH,), jnp.int32) for _ in range(NSB)]
        + [pltpu.VMEM((CH,), jnp.float32) for _ in range(NSB)]
        + [pltpu.VMEM((CH, D), jnp.float32) for _ in range(NSB)]
        + [pltpu.VMEM((2000,), jnp.float32)]
        + [pltpu.VMEM_SHARED((N, D), jnp.float32),
           pltpu.VMEM_SHARED((N,), jnp.float32)]
        + [pltpu.SemaphoreType.DMA] * (4 * NSB)
    ),
)
def _scatter(m_hbm, at_hbm, row_hbm, num_out, asum_out, *rest):
    idx_b = rest[:NSB]
    at_b = rest[NSB:2 * NSB]
    m_b = rest[2 * NSB:3 * NSB]
    zb = rest[3 * NSB]
    num_s, as_s = rest[3 * NSB + 1:3 * NSB + 3]
    sems = rest[3 * NSB + 3:]
    sem_i = sems[:NSB]
    sem_m = sems[NSB:2 * NSB]
    sem_a = sems[2 * NSB:3 * NSB]
    sem_s = sems[3 * NSB:]
    cid = lax.axis_index("c")
    sid = lax.axis_index("s")
    wid = sid * NC + cid
    ebase = wid * EPW

    zero16 = jnp.zeros((16,), jnp.float32)

    def zm(i, carry):
        m_b[0][i // 8, pl.ds((i % 8) * 16, 16)] = zero16
        return carry

    lax.fori_loop(0, CH * 8, zm, 0)

    def za(i, carry):
        zb[pl.ds(i * 16, 16)] = zero16
        return carry

    lax.fori_loop(0, 125, za, 0)

    for t in range(8):
        c = sid + t * NS

        @pl.when(c < NZCH)
        def _():
            pltpu.sync_copy(m_b[0], num_s.at[pl.ds(c * CH, CH)])

    @pl.when(sid == 0)
    def _():
        for q in range(5):
            pltpu.sync_copy(zb, as_s.at[pl.ds(q * 2000, 2000)])

    plsc.subcore_barrier()

    def _start_loads(b, j):
        pltpu.async_copy(row_hbm.at[wid, j, 0], idx_b[b], sem_i[b])
        pltpu.async_copy(m_hbm.at[pl.ds(ebase + j * CH, CH)], m_b[b], sem_m[b])
        pltpu.async_copy(at_hbm.at[wid, j, 0], at_b[b], sem_a[b])

    def _wait_loads(b, j):
        pltpu.make_async_copy(row_hbm.at[wid, j, 0], idx_b[b], sem_i[b]).wait()
        pltpu.make_async_copy(m_hbm.at[pl.ds(ebase + j * CH, CH)], m_b[b],
                              sem_m[b]).wait()
        pltpu.make_async_copy(at_hbm.at[wid, j, 0], at_b[b], sem_a[b]).wait()

    def _process(b):
        hm = pltpu.async_copy(m_b[b], num_s.at[idx_b[b]], sem_s[b], add=True)
        ha = pltpu.async_copy(at_b[b], as_s.at[idx_b[b]], sem_s[b], add=True)
        hm.wait()
        ha.wait()

    for b in range(NSB):
        _start_loads(b, b)

    def body(jo, carry):
        j = jo * NSB
        for b in range(NSB):
            _wait_loads(b, j + b)
            _process(b)

            @pl.when(j + b + NSB < NCH)
            def _():
                _start_loads(b, j + b + NSB)

        return carry

    lax.fori_loop(0, NCH // NSB, body, 0)
    _wait_loads(0, NCH - 1)
    _process(0)
    plsc.subcore_barrier()

    @pl.when(sid == 0)
    def _():
        pltpu.sync_copy(as_s, asum_out.at[cid, 0])

    for t in range(8):
        c = sid + t * NS

        @pl.when(c < NZCH)
        def _():
            r = c * CH
            pltpu.sync_copy(num_s.at[pl.ds(r, CH)], num_out.at[cid, pl.ds(r, CH)])


def _norm_body(num_ref, asum_ref, out_ref):
    nm = num_ref[0] + num_ref[1]
    norm = jnp.sum(asum_ref[...], axis=0)
    safe = jnp.where(norm > 0, norm, 1.0)
    out_ref[...] = jnp.where(norm > 0, nm / safe, nm)


def _norm(num_p, asum_p):
    bn = 1000
    return pl.pallas_call(
        _norm_body,
        grid=(N // bn,),
        in_specs=[
            pl.BlockSpec((NC, bn, D), lambda i: (0, i, 0)),
            pl.BlockSpec((NC, bn, 1), lambda i: (0, i, 0)),
        ],
        out_specs=pl.BlockSpec((bn, D), lambda i: (i, 0)),
        out_shape=jax.ShapeDtypeStruct((N, D), jnp.float32),
    )(num_p, asum_p)


def kernel(hidden_atom_emb, edge_emb, edge_list, W1, b1, W2, b2, linear_coef):
    col3 = edge_list[:, 1].astype(jnp.int32).reshape(NW, NCH, 1, CH)
    row3 = edge_list[:, 0].astype(jnp.int32).reshape(NW, NCH, 1, CH)
    g = _gather(hidden_atom_emb, col3)
    m, attn = _mlp(
        edge_emb, g, W1,
        b1.reshape(1, D), W2, b2.reshape(1, D), linear_coef.reshape(1, D),
    )
    at3 = attn.reshape(NW, NCH, 1, CH)
    num_p, asum_p = _scatter(m, at3, row3)
    asum = asum_p.reshape(NC, N, 1)
    return _norm(num_p, asum)


# speedup vs baseline: 5.1324x; 1.0370x over previous
import functools

import jax
import jax.numpy as jnp
from jax import lax
from jax.experimental import pallas as pl
from jax.experimental.pallas import tpu as pltpu
from jax.experimental.pallas import tpu_sc as plsc

N = 10000
E = 320000
D = 128
ED = 16
NC = 2
NS = 16
NW = NC * NS
EPW = E // NW
CH = 80
NCH = EPW // CH
NZCH = N // CH
NB = 80
NGB = 4
NSB = 4

_mesh = plsc.VectorSubcoreMesh(core_axis_name="c", subcore_axis_name="s")


@functools.partial(
    pl.kernel,
    mesh=_mesh,
    out_type=jax.ShapeDtypeStruct((E, D), jnp.float32),
    scratch_types=(
        [pltpu.VMEM((CH,), jnp.int32) for _ in range(NGB)]
        + [pltpu.VMEM((CH, D), jnp.float32) for _ in range(NGB)]
        + [pltpu.VMEM_SHARED((N, D), jnp.float32)]
        + [pltpu.SemaphoreType.DMA] * (3 * NGB)
    ),
)
def _gather(table_hbm, col_hbm, out_hbm, *rest):
    idx_b = rest[:NGB]
    g_b = rest[NGB:2 * NGB]
    table_s = rest[2 * NGB]
    sems = rest[2 * NGB + 1:]
    sem_i = sems[:NGB]
    sem_g = sems[NGB:2 * NGB]
    sem_w = sems[2 * NGB:]
    sid = lax.axis_index("s")
    wid = sid * NC + lax.axis_index("c")
    ebase = wid * EPW

    for t in range(8):
        c = sid + t * NS

        @pl.when(c < NZCH)
        def _():
            pltpu.sync_copy(table_hbm.at[pl.ds(c * CH, CH)],
                            table_s.at[pl.ds(c * CH, CH)])

    plsc.subcore_barrier()

    def _start_idx(b, j):
        pltpu.async_copy(col_hbm.at[wid, j, 0], idx_b[b], sem_i[b])

    def _wait_idx(b, j):
        pltpu.make_async_copy(col_hbm.at[wid, j, 0], idx_b[b], sem_i[b]).wait()

    def _wait_write(b):
        pltpu.make_async_copy(g_b[b], out_hbm.at[pl.ds(ebase, CH)],
                              sem_w[b]).wait()

    for b in range(NGB):
        _start_idx(b, b)

    def body(jo, carry):
        j = jo * NGB
        for b in range(NGB):
            jj = j + b

            @pl.when(jj >= NGB)
            def _():
                _wait_write(b)

            _wait_idx(b, jj)
            pltpu.async_copy(table_s.at[idx_b[b]], g_b[b], sem_g[b]).wait()
            pltpu.async_copy(g_b[b], out_hbm.at[pl.ds(ebase + jj * CH, CH)],
                             sem_w[b])

            @pl.when(jj + NGB < NCH)
            def _():
                _start_idx(b, jj + NGB)

        return carry

    lax.fori_loop(0, NCH // NGB, body, 0)
    _wait_write(0)
    _wait_idx(0, NCH - 1)
    pltpu.async_copy(table_s.at[idx_b[0]], g_b[0], sem_g[0]).wait()
    pltpu.async_copy(g_b[0], out_hbm.at[pl.ds(ebase + (NCH - 1) * CH, CH)],
                     sem_w[0])
    for b in range(NGB):
        _wait_write(b)


def _mlp_body(e_ref, g_ref, w1_ref, b1_ref, w2_ref, b2_ref, c_ref,
              m_ref, a_ref):
    h = jnp.tanh(
        jnp.dot(e_ref[...], w1_ref[...], preferred_element_type=jnp.float32)
        + b1_ref[...]
    )
    f = jnp.dot(h, w2_ref[...], preferred_element_type=jnp.float32) + b2_ref[...]
    hce = g_ref[...] * f
    attn = jnp.exp(jnp.sum(hce * c_ref[...], axis=1, keepdims=True))
    m_ref[...] = hce * attn
    a_ref[...] = attn


def _mlp(edge_emb, g, W1, b1r, W2, b2r, cr):
    be = 8000
    grid = E // be
    return pl.pallas_call(
        _mlp_body,
        grid=(grid,),
        in_specs=[
            pl.BlockSpec((be, ED), lambda i: (i, 0)),
            pl.BlockSpec((be, D), lambda i: (i, 0)),
            pl.BlockSpec((ED, D), lambda i: (0, 0)),
            pl.BlockSpec((1, D), lambda i: (0, 0)),
            pl.BlockSpec((D, D), lambda i: (0, 0)),
            pl.BlockSpec((1, D), lambda i: (0, 0)),
            pl.BlockSpec((1, D), lambda i: (0, 0)),
        ],
        out_specs=[
            pl.BlockSpec((be, D), lambda i: (i, 0)),
            pl.BlockSpec((be, 1), lambda i: (i, 0)),
        ],
        out_shape=[
            jax.ShapeDtypeStruct((E, D), jnp.float32),
            jax.ShapeDtypeStruct((E, 1), jnp.float32),
        ],
    )(edge_emb, g, W1, b1r, W2, b2r, cr)


@functools.partial(
    pl.kernel,
    mesh=_mesh,
    out_type=[
        jax.ShapeDtypeStruct((NC, N, D), jnp.float32),
        jax.ShapeDtypeStruct((NC, 1, N), jnp.float32),
    ],
    scratch_types=(
        [pltpu.VMEM((CH,), jnp.int32) for _ in range(NSB)]
        + [pltpu.VMEM((CH,), jnp.float32) for _ in range(NSB)]
        + [pltpu.VMEM((CH, D), jnp.float32) for _ in range(NSB)]
        + [pltpu.VMEM((2000,), jnp.float32)]
        + [pltpu.VMEM_SHARED((N, D), jnp.float32),
           pltpu.VMEM_SHARED((N,), jnp.float32)]
        + [pltpu.SemaphoreType.DMA] * (4 * NSB)
    ),
)
def _scatter(m_hbm, at_hbm, row_hbm, num_out, asum_out, *rest):
    idx_b = rest[:NSB]
    at_b = rest[NSB:2 * NSB]
    m_b = rest[2 * NSB:3 * NSB]
    zb = rest[3 * NSB]
    num_s, as_s = rest[3 * NSB + 1:3 * NSB + 3]
    sems = rest[3 * NSB + 3:]
    sem_i = sems[:NSB]
    sem_m = sems[NSB:2 * NSB]
    sem_a = sems[2 * NSB:3 * NSB]
    sem_s = sems[3 * NSB:]
    cid = lax.axis_index("c")
    sid = lax.axis_index("s")
    wid = sid * NC + cid
    ebase = wid * EPW

    zero16 = jnp.zeros((16,), jnp.float32)

    def zm(i, carry):
        m_b[0][i // 8, pl.ds((i % 8) * 16, 16)] = zero16
        return carry

    lax.fori_loop(0, CH * 8, zm, 0)

    def za(i, carry):
        zb[pl.ds(i * 16, 16)] = zero16
        return carry

    lax.fori_loop(0, 125, za, 0)

    for t in range(8):
        c = sid + t * NS

        @pl.when(c < NZCH)
        def _():
            pltpu.sync_copy(m_b[0], num_s.at[pl.ds(c * CH, CH)])

    @pl.when(sid == 0)
    def _():
        for q in range(5):
            pltpu.sync_copy(zb, as_s.at[pl.ds(q * 2000, 2000)])

    plsc.subcore_barrier()

    def _start_loads(b, j):
        pltpu.async_copy(row_hbm.at[wid, j, 0], idx_b[b], sem_i[b])
        pltpu.async_copy(m_hbm.at[pl.ds(ebase + j * CH, CH)], m_b[b], sem_m[b])
        pltpu.async_copy(at_hbm.at[wid, j, 0], at_b[b], sem_a[b])

    def _wait_loads(b, j):
        pltpu.make_async_copy(row_hbm.at[wid, j, 0], idx_b[b], sem_i[b]).wait()
        pltpu.make_async_copy(m_hbm.at[pl.ds(ebase + j * CH, CH)], m_b[b],
                              sem_m[b]).wait()
        pltpu.make_async_copy(at_hbm.at[wid, j, 0], at_b[b], sem_a[b]).wait()

    def _process(b):
        hm = pltpu.async_copy(m_b[b], num_s.at[idx_b[b]], sem_s[b], add=True)
        ha = pltpu.async_copy(at_b[b], as_s.at[idx_b[b]], sem_s[b], add=True)
        hm.wait()
        ha.wait()

    for b in range(NSB):
        _start_loads(b, b)

    def body(jo, carry):
        j = jo * NSB
        for b in range(NSB):
            _wait_loads(b, j + b)
            _process(b)

            @pl.when(j + b + NSB < NCH)
            def _():
                _start_loads(b, j + b + NSB)

        return carry

    lax.fori_loop(0, NCH // NSB, body, 0)
    _wait_loads(0, NCH - 1)
    _process(0)
    plsc.subcore_barrier()

    @pl.when(sid == 0)
    def _():
        pltpu.sync_copy(as_s, asum_out.at[cid, 0])

    for t in range(8):
        c = sid + t * NS

        @pl.when(c < NZCH)
        def _():
            r = c * CH
            pltpu.sync_copy(num_s.at[pl.ds(r, CH)], num_out.at[cid, pl.ds(r, CH)])


def _norm_body(num_ref, asum_ref, out_ref):
    nm = num_ref[0] + num_ref[1]
    norm = jnp.sum(asum_ref[...], axis=0)
    safe = jnp.where(norm > 0, norm, 1.0)
    out_ref[...] = jnp.where(norm > 0, nm / safe, nm)


def _norm(num_p, asum_p):
    bn = 1000
    return pl.pallas_call(
        _norm_body,
        grid=(N // bn,),
        in_specs=[
            pl.BlockSpec((NC, bn, D), lambda i: (0, i, 0)),
            pl.BlockSpec((NC, bn, 1), lambda i: (0, i, 0)),
        ],
        out_specs=pl.BlockSpec((bn, D), lambda i: (i, 0)),
        out_shape=jax.ShapeDtypeStruct((N, D), jnp.float32),
    )(num_p, asum_p)


def kernel(hidden_atom_emb, edge_emb, edge_list, W1, b1, W2, b2, linear_coef):
    col3 = edge_list[:, 1].astype(jnp.int32).reshape(NW, NCH, 1, CH)
    row3 = edge_list[:, 0].astype(jnp.int32).reshape(NW, NCH, 1, CH)
    g = _gather(hidden_atom_emb, col3)
    m, attn = _mlp(
        edge_emb, g, W1,
        b1.reshape(1, D), W2, b2.reshape(1, D), linear_coef.reshape(1, D),
    )
    at3 = attn.reshape(NW, NCH, 1, CH)
    num_p, asum_p = _scatter(m, at3, row3)
    asum = asum_p.reshape(NC, N, 1)
    return _norm(num_p, asum)
